# baseline (speedup 1.0000x reference)
import jax
import jax.numpy as jnp
from jax.experimental import pallas as pl
from jax.experimental.pallas import tpu as pltpu

NUM_N = 10
IN_DIM = 784


def _kern(img_ref, syn_ref, out_ref):
    img = img_ref[...]
    basal = jnp.where(img > 0.5, 1.0, 0.0)
    syn = syn_ref[...]
    firing = jnp.sum(syn * basal, axis=1, keepdims=True)
    m = jnp.max(firing)
    idxs = jax.lax.broadcasted_iota(jnp.int32, firing.shape, 0)
    best = jnp.min(jnp.where(firing >= m, idxs, NUM_N))
    out_ref[0] = best


def kernel(image, synapses):
    img2d = image.reshape(1, IN_DIM)
    syn2d = synapses.reshape(NUM_N, IN_DIM)
    out = pl.pallas_call(
        _kern,
        out_shape=jax.ShapeDtypeStruct((1,), jnp.int32),
        in_specs=[
            pl.BlockSpec(memory_space=pltpu.VMEM),
            pl.BlockSpec(memory_space=pltpu.VMEM),
        ],
        out_specs=pl.BlockSpec(memory_space=pltpu.SMEM),
    )(img2d, syn2d)
    return out[0]
